# Initial kernel scaffold; baseline (speedup 1.0000x reference)
import jax
import jax.numpy as jnp
from jax.experimental import pallas as pl


def kernel(q_sub, q_rel, hidden, edges, nodes, old_nodes_new_idx, rela_embed, Ws_attn, Wr_attn, Wqr_attn_w, Wqr_attn_b, w_alpha_w, w_alpha_b, W_h):
    raise NotImplementedError("write your pallas kernel here")


# speedup vs baseline: 4.4449x; 4.4449x over previous
import functools

import jax
import jax.numpy as jnp
from jax import lax
from jax.experimental import pallas as pl
from jax.experimental.pallas import tpu as pltpu
from jax.experimental.pallas import tpu_sc as plsc

N_NODE = 10000
IN_DIM = 128
E = 320000
B_Q = 16384
REL_PAD = 10240

NC = 2
NS = 16
NW = NC * NS
EDGES_PER_TILE = E // NW
CHUNK = 80
NCHUNKS = EDGES_PER_TILE // CHUNK
QP_TILE = B_Q // NW
ROWS_A = 632
ROWS_B = N_NODE - 15 * ROWS_A

_SC_MESH = plsc.VectorSubcoreMesh(core_axis_name="c", subcore_axis_name="s")


def _stab_body(h_ref, ws_ref, out_ref):
    out_ref[...] = lax.dot_general(
        h_ref[...], ws_ref[...], (((1,), (1,)), ((), ())),
        preferred_element_type=jnp.float32)


def _rtab_body(r_ref, wr_ref, wqr_ref, b_ref, ar_ref, aq_ref):
    r = r_ref[...]
    ar_ref[...] = lax.dot_general(
        r, wr_ref[...], (((1,), (1,)), ((), ())),
        preferred_element_type=jnp.float32)
    aq_ref[...] = lax.dot_general(
        r, wqr_ref[...], (((1,), (1,)), ((), ())),
        preferred_element_type=jnp.float32) + b_ref[...]


def _final_body(p_ref, wh_ref, out_ref):
    s = p_ref[0] + p_ref[1]
    out_ref[...] = lax.dot_general(
        s, wh_ref[...], (((1,), (1,)), ((), ())),
        preferred_element_type=jnp.float32)


def _sc_aq_body(aqf_hbm, qrel_hbm, out_hbm, idx_v, rows_v, sem):
    cid = lax.axis_index("c")
    sid = lax.axis_index("s")
    wid = sid * NC + cid
    base = wid * QP_TILE

    for k in range(QP_TILE // 128):
        pltpu.sync_copy(qrel_hbm.at[pl.ds(base + 128 * k, 128)], idx_v)
        pltpu.async_copy(aqf_hbm.at[idx_v], rows_v, sem).wait()
        pltpu.sync_copy(rows_v, out_hbm.at[pl.ds(base + 128 * k, 128)])


_sc_aq = functools.partial(
    pl.kernel,
    out_type=jax.ShapeDtypeStruct((B_Q, IN_DIM), jnp.float32),
    mesh=_SC_MESH,
    scratch_types=[
        pltpu.VMEM((128,), jnp.int32),
        pltpu.VMEM((128, IN_DIM), jnp.float32),
        pltpu.SemaphoreType.DMA,
    ],
)(_sc_aq_body)


def _sc_edge_body(cols_hbm, as_hbm, ar_hbm, aqb_hbm, hid_hbm, rel_hbm,
                  wv_hbm, out_hbm,
                  cols_v, sub_v, rel_v, ridx_v, obj_v,
                  buf_a, buf_b, buf_c, msg, zbuf, wv_v, acc,
                  sem1, sem2, sem3):
    cid = lax.axis_index("c")
    sid = lax.axis_index("s")
    wid = sid * NC + cid

    pltpu.sync_copy(wv_hbm, wv_v)

    zero16 = jnp.zeros((16,), jnp.float32)
    for i in range(8):
        for j in range(8):
            zbuf[i, pl.ds(16 * j, 16)] = zero16
    row0 = sid * ROWS_A
    nz = jnp.where(sid < 15, ROWS_A // 8, ROWS_B // 8)

    def _zacc(z, carry):
        pltpu.sync_copy(zbuf, acc.at[pl.ds(row0 + z * 8, 8)])
        return carry

    lax.fori_loop(0, nz, _zacc, 0)
    plsc.subcore_barrier()

    w_regs = [wv_v[pl.ds(16 * j, 16)] for j in range(8)]
    wb_vec = wv_v[pl.ds(IN_DIM, 16)]

    def _chunk(ci, carry):
        pltpu.sync_copy(cols_hbm.at[wid, ci], cols_v)
        for k in range(CHUNK // 16):
            sl = pl.ds(16 * k, 16)
            sub_v[sl] = cols_v[0, sl]
            rel_v[sl] = cols_v[1, sl]
            ridx_v[sl] = cols_v[2, sl]
            obj_v[sl] = cols_v[3, sl]

        cp1 = pltpu.async_copy(as_hbm.at[sub_v], buf_a, sem1)
        cp2 = pltpu.async_copy(ar_hbm.at[rel_v], buf_b, sem2)
        cp3 = pltpu.async_copy(aqb_hbm.at[ridx_v], buf_c, sem3)
        cp1.wait()
        cp2.wait()
        cp3.wait()

        def _attn(e, icarry):
            accv = wb_vec
            for j in range(8):
                sl = pl.ds(16 * j, 16)
                t = buf_a[e, sl] + buf_b[e, sl] + buf_c[e, sl]
                t = jnp.maximum(t, 0.0)
                accv = accv + t * w_regs[j]
            ssum = jnp.sum(accv)
            sv = jnp.full((16,), ssum, jnp.float32)
            av = 1.0 / (1.0 + jnp.exp(-sv))
            msg[e, pl.ds(0, 16)] = av
            return icarry

        lax.fori_loop(0, CHUNK, _attn, 0)

        cp4 = pltpu.async_copy(hid_hbm.at[sub_v], buf_a, sem1)
        cp5 = pltpu.async_copy(rel_hbm.at[rel_v], buf_b, sem2)
        cp4.wait()
        cp5.wait()

        def _msg(e, icarry):
            av = msg[e, pl.ds(0, 16)]
            for j in range(8):
                sl = pl.ds(16 * j, 16)
                msg[e, sl] = (buf_a[e, sl] + buf_b[e, sl]) * av
            return icarry

        lax.fori_loop(0, CHUNK, _msg, 0)
        pltpu.sync_copy(msg, acc.at[obj_v], add=True)
        return carry

    lax.fori_loop(0, NCHUNKS, _chunk, 0)
    plsc.subcore_barrier()

    @pl.when(sid < 15)
    def _():
        pltpu.sync_copy(acc.at[pl.ds(sid * ROWS_A, ROWS_A)],
                        out_hbm.at[cid, pl.ds(sid * ROWS_A, ROWS_A)])

    @pl.when(sid == 15)
    def _():
        pltpu.sync_copy(acc.at[pl.ds(15 * ROWS_A, ROWS_B)],
                        out_hbm.at[cid, pl.ds(15 * ROWS_A, ROWS_B)])


_sc_edge = functools.partial(
    pl.kernel,
    out_type=jax.ShapeDtypeStruct((NC, N_NODE, IN_DIM), jnp.float32),
    mesh=_SC_MESH,
    compiler_params=pltpu.CompilerParams(needs_layout_passes=False),
    scratch_types=[
        pltpu.VMEM((4, CHUNK), jnp.int32),
        pltpu.VMEM((CHUNK,), jnp.int32),
        pltpu.VMEM((CHUNK,), jnp.int32),
        pltpu.VMEM((CHUNK,), jnp.int32),
        pltpu.VMEM((CHUNK,), jnp.int32),
        pltpu.VMEM((CHUNK, IN_DIM), jnp.float32),
        pltpu.VMEM((CHUNK, IN_DIM), jnp.float32),
        pltpu.VMEM((CHUNK, IN_DIM), jnp.float32),
        pltpu.VMEM((CHUNK, IN_DIM), jnp.float32),
        pltpu.VMEM((8, IN_DIM), jnp.float32),
        pltpu.VMEM((144,), jnp.float32),
        pltpu.VMEM_SHARED((N_NODE, IN_DIM), jnp.float32),
        pltpu.SemaphoreType.DMA,
        pltpu.SemaphoreType.DMA,
        pltpu.SemaphoreType.DMA,
    ],
)(_sc_edge_body)


def kernel(q_sub, q_rel, hidden, edges, nodes, old_nodes_new_idx,
           rela_embed, Ws_attn, Wr_attn, Wqr_attn_w, Wqr_attn_b,
           w_alpha_w, w_alpha_b, W_h):
    cols = jnp.stack(
        [edges[:, 4], edges[:, 2], edges[:, 0], edges[:, 5]], axis=0)
    cols = cols.reshape(4, NW, NCHUNKS, CHUNK).transpose(1, 2, 0, 3)

    as_tab = pl.pallas_call(
        _stab_body,
        grid=(N_NODE // 400,),
        in_specs=[
            pl.BlockSpec((400, IN_DIM), lambda i: (i, 0)),
            pl.BlockSpec((IN_DIM, IN_DIM), lambda i: (0, 0)),
        ],
        out_specs=pl.BlockSpec((400, IN_DIM), lambda i: (i, 0)),
        out_shape=jax.ShapeDtypeStruct((N_NODE, IN_DIM), jnp.float32),
    )(hidden, Ws_attn)

    rel_pad = jnp.pad(rela_embed, ((0, REL_PAD - rela_embed.shape[0]), (0, 0)))
    ar_tab, aqf = pl.pallas_call(
        _rtab_body,
        grid=(REL_PAD // 256,),
        in_specs=[
            pl.BlockSpec((256, IN_DIM), lambda i: (i, 0)),
            pl.BlockSpec((IN_DIM, IN_DIM), lambda i: (0, 0)),
            pl.BlockSpec((IN_DIM, IN_DIM), lambda i: (0, 0)),
            pl.BlockSpec((1, IN_DIM), lambda i: (0, 0)),
        ],
        out_specs=[
            pl.BlockSpec((256, IN_DIM), lambda i: (i, 0)),
            pl.BlockSpec((256, IN_DIM), lambda i: (i, 0)),
        ],
        out_shape=[
            jax.ShapeDtypeStruct((REL_PAD, IN_DIM), jnp.float32),
            jax.ShapeDtypeStruct((REL_PAD, IN_DIM), jnp.float32),
        ],
    )(rel_pad, Wr_attn, Wqr_attn_w, Wqr_attn_b.reshape(1, IN_DIM))

    aq_b = _sc_aq(aqf, q_rel.astype(jnp.int32))

    wv = jnp.concatenate([
        w_alpha_w.reshape(-1), w_alpha_b.reshape(-1),
        jnp.zeros((15,), jnp.float32)])

    partials = _sc_edge(cols, as_tab, ar_tab, aq_b, hidden, rel_pad, wv)

    out = pl.pallas_call(
        _final_body,
        grid=(N_NODE // 400,),
        in_specs=[
            pl.BlockSpec((NC, 400, IN_DIM), lambda i: (0, i, 0)),
            pl.BlockSpec((IN_DIM, IN_DIM), lambda i: (0, 0)),
        ],
        out_specs=pl.BlockSpec((400, IN_DIM), lambda i: (i, 0)),
        out_shape=jax.ShapeDtypeStruct((N_NODE, IN_DIM), jnp.float32),
    )(partials, W_h)
    return out
